# baseline (speedup 1.0000x reference)
import dataclasses
import functools

import jax
import jax.numpy as jnp
from jax import lax
from jax.experimental import pallas as pl
from jax.experimental.pallas import tpu as pltpu
from jax.experimental.pallas import tpu_sc as plsc

N = 10000
E = 320000
D = 128
G = 64
DOUT = 10

NC = 2
NS = 16
NW = NC * NS
EPW = E // NW
CH = 128
NCH = -(-EPW // CH)
BK = 8
NBW = 10
NB = 12
EPADB = NB * BK * CH
EPAD = NCH * CH
NP = 10112
RPS = NP // NS

_F32 = jnp.float32

_mesh = plsc.VectorSubcoreMesh(core_axis_name="c", subcore_axis_name="s")

_no_layout_cp = pltpu.CompilerParams()
if "needs_layout_passes" in pltpu.CompilerParams.__dataclass_fields__:
    _no_layout_cp = dataclasses.replace(_no_layout_cp,
                                        needs_layout_passes=False)


@functools.partial(
    pl.kernel,
    mesh=_mesh,
    out_type=jax.ShapeDtypeStruct((NW, NP), _F32),
    compiler_params=_no_layout_cp,
    scratch_types=[
        pltpu.VMEM((NCH, CH), jnp.int32),
        pltpu.VMEM((NP,), _F32),
    ],
)
def _deg_sc(dst_hbm, zeros_hbm, out_hbm, dstv, hist):
    c = lax.axis_index("c")
    s = lax.axis_index("s")
    wid = c * NS + s
    pltpu.sync_copy(zeros_hbm, hist)
    pltpu.sync_copy(dst_hbm.at[wid], dstv)
    ones16 = jnp.full((16,), 1.0, _F32)

    @pl.loop(0, NCH)
    def _(j):
        for k in range(CH // 16):
            d16 = dstv[j, pl.ds(k * 16, 16)]
            plsc.addupdate_scatter(hist, [d16], ones16)

    pltpu.sync_copy(hist, out_hbm.at[wid])


@functools.partial(
    pl.kernel,
    mesh=_mesh,
    out_type=jax.ShapeDtypeStruct((NC, NP, D), _F32),
    scratch_types=[
        pltpu.VMEM((BK, 2, CH), jnp.int32),
        pltpu.VMEM((BK, 2, CH), jnp.int32),
        pltpu.VMEM((CH, D), _F32),
        pltpu.VMEM((CH, D), _F32),
        pltpu.VMEM_SHARED((NP, D), _F32),
        pltpu.SemaphoreType.DMA,
        pltpu.SemaphoreType.DMA,
        pltpu.SemaphoreType.DMA,
        pltpu.SemaphoreType.DMA,
        pltpu.SemaphoreType.DMA,
        pltpu.SemaphoreType.DMA,
    ],
)
def _gs_sc(p_hbm, sidx_hbm, zeros_hbm, out_hbm,
           ib0, ib1, b0, b1, acc, gsem0, gsem1, isem0, isem1, ssem0, ssem1):
    c = lax.axis_index("c")
    s = lax.axis_index("s")
    wid = c * NS + s
    r0 = s * RPS
    pltpu.sync_copy(zeros_hbm.at[pl.ds(r0, RPS)], acc.at[pl.ds(r0, RPS)])
    plsc.subcore_barrier()

    def _ibload(mb, ib, sem):
        return pltpu.make_async_copy(sidx_hbm.at[wid, mb], ib, sem)

    def _g(ib, k, buf, sem):
        return pltpu.make_async_copy(p_hbm.at[ib.at[k, 0]], buf, sem)

    class _scat:
        def __init__(self, buf, ib, k, sem):
            self.buf, self.ib, self.k, self.sem = buf, ib, k, sem

        def start(self):
            pltpu.async_copy(self.buf, acc.at[self.ib.at[self.k, 1]],
                             self.sem, add=True)

        def wait(self):
            pltpu.make_async_copy(self.buf, acc.at[self.ib.at[self.k, 1]],
                                  self.sem).wait()

    def _process(cur, nxt, sem_nxt, sem_cur, m_reload):
        for k in range(BK):
            buf, gsem, ssem = ((b0, gsem0, ssem0) if k % 2 == 0
                               else (b1, gsem1, ssem1))
            nbuf, ngsem = (b1, gsem1) if k % 2 == 0 else (b0, gsem0)
            _g(cur, k, buf, gsem).wait()
            sc = _scat(buf, cur, k, ssem)
            sc.start()
            if k < BK - 1:
                _g(cur, k + 1, nbuf, ngsem).start()
            else:
                _ibload(m_reload, nxt, sem_nxt).wait()
                _g(nxt, 0, nbuf, ngsem).start()
            sc.wait()
        _ibload(m_reload, cur, sem_cur).start()

    _ibload(0, ib0, isem0).start()
    _ibload(0, ib0, isem0).wait()
    _g(ib0, 0, b0, gsem0).start()
    _ibload(1, ib1, isem1).start()

    @pl.loop(0, NBW, step=2)
    def _(m):
        _process(ib0, ib1, isem1, isem0, m + 2)
        _process(ib1, ib0, isem0, isem1, m + 3)

    _g(ib0, 0, b0, gsem0).wait()
    _ibload(NB - 1, ib1, isem1).wait()

    plsc.subcore_barrier()
    pltpu.sync_copy(acc.at[pl.ds(r0, RPS)], out_hbm.at[c, pl.ds(r0, RPS)])


def _tc_mm(x, w):
    def body(x_ref, w_ref, o_ref):
        o_ref[...] = jnp.dot(x_ref[...], w_ref[...],
                             preferred_element_type=_F32,
                             precision=lax.Precision.HIGHEST)

    return pl.pallas_call(
        body, out_shape=jax.ShapeDtypeStruct((x.shape[0], w.shape[1]), _F32),
    )(x, w)


def _tc_scale(cnt, h1):
    def body(cnt_ref, h_ref, dinv_ref, p_ref):
        ones32 = jnp.ones((NW, 1), _F32)
        degc = lax.dot_general(cnt_ref[...], ones32, (((0,), (0,)), ((), ())),
                               preferred_element_type=_F32,
                               precision=lax.Precision.HIGHEST)
        deg = degc[0:N, :] + 1.0
        dinv = lax.rsqrt(deg)
        dinv_ref[...] = dinv
        p_ref[0:N, :] = h_ref[...] * dinv
        p_ref[N:NP, :] = jnp.zeros((NP - N, D), _F32)

    return pl.pallas_call(
        body,
        out_shape=[jax.ShapeDtypeStruct((N, 1), _F32),
                   jax.ShapeDtypeStruct((NP, D), _F32)],
    )(cnt, h1)


def _tc_mid(s1, p1, dinv, b, w):
    def body(s_ref, p_ref, dinv_ref, b_ref, w_ref, p2_ref):
        u = s_ref[0, 0:N, :] + s_ref[1, 0:N, :] + p_ref[0:N, :]
        h = jnp.maximum(u * dinv_ref[...] + b_ref[...], 0.0)
        h2 = jnp.dot(h, w_ref[...], preferred_element_type=_F32,
                     precision=lax.Precision.HIGHEST)
        p2_ref[0:N, :] = h2 * dinv_ref[...]
        p2_ref[N:NP, :] = jnp.zeros((NP - N, D), _F32)

    return pl.pallas_call(
        body, out_shape=jax.ShapeDtypeStruct((NP, D), _F32),
    )(s1, p1, dinv, b, w)


def _tc_final(s2, p2, dinv, b, batch2, fc_w, fc_b):
    def body(s_ref, p_ref, dinv_ref, b_ref, batch_ref, fcw_ref, fcb_ref,
             o_ref):
        u = s_ref[0, 0:N, :] + s_ref[1, 0:N, :] + p_ref[0:N, :]
        h = jnp.maximum(u * dinv_ref[...] + b_ref[...], 0.0)
        gids = lax.broadcasted_iota(jnp.int32, (N, G), 1)
        m = (batch_ref[...] == gids).astype(_F32)
        gsum = lax.dot_general(m, h, (((0,), (0,)), ((), ())),
                               preferred_element_type=_F32,
                               precision=lax.Precision.HIGHEST)
        counts = jnp.sum(m, axis=0)[:, None]
        mean = gsum / jnp.maximum(counts, 1.0)
        logits = jnp.dot(mean, fcw_ref[...], preferred_element_type=_F32,
                         precision=lax.Precision.HIGHEST) + fcb_ref[...]
        mx = jnp.max(logits, axis=1, keepdims=True)
        lse = jnp.log(jnp.sum(jnp.exp(logits - mx), axis=1,
                              keepdims=True)) + mx
        o_ref[...] = logits - lse

    return pl.pallas_call(
        body, out_shape=jax.ShapeDtypeStruct((G, DOUT), _F32),
    )(s2, p2, dinv, b, batch2, fc_w, fc_b)


def kernel(x, edge_index, batch, W1, b1, W2, b2, fc_W, fc_b):
    ei = edge_index.astype(jnp.int32)
    pad = jnp.full((NW, EPAD - EPW), N, jnp.int32)
    dst3 = jnp.concatenate([ei[1].reshape(NW, EPW), pad], axis=1)
    dst3 = dst3.reshape(NW, NCH, CH)
    padb = jnp.full((NW, EPADB - EPW), N, jnp.int32)
    srcb = jnp.concatenate([ei[0].reshape(NW, EPW), padb], axis=1)
    dstb = jnp.concatenate([ei[1].reshape(NW, EPW), padb], axis=1)
    sidx = jnp.stack([srcb.reshape(NW, NB * BK, CH),
                      dstb.reshape(NW, NB * BK, CH)], axis=2)
    sidx = sidx.reshape(NW, NB, BK, 2, CH)

    zerosN = jnp.zeros((NP,), _F32)
    zerosD = jnp.zeros((NP, D), _F32)
    batch2 = batch.astype(jnp.int32).reshape(N, 1)
    b1r = b1.reshape(1, D)
    b2r = b2.reshape(1, D)
    fcbr = fc_b.reshape(1, DOUT)

    cnt = _deg_sc(dst3, zerosN)
    h1 = _tc_mm(x, W1)
    dinv, p1 = _tc_scale(cnt, h1)
    s1 = _gs_sc(p1, sidx, zerosD)
    p2 = _tc_mid(s1, p1, dinv, b1r, W2)
    s2 = _gs_sc(p2, sidx, zerosD)
    return _tc_final(s2, p2, dinv, b2r, batch2, fc_W, fcbr)


# speedup vs baseline: 2.0215x; 2.0215x over previous
import dataclasses
import functools

import jax
import jax.numpy as jnp
from jax import lax
from jax.experimental import pallas as pl
from jax.experimental.pallas import tpu as pltpu
from jax.experimental.pallas import tpu_sc as plsc

N = 10000
E = 320000
D = 128
G = 64
DOUT = 10

NC = 2
NS = 16
NW = NC * NS
EPW = E // NW
CH = 128
NCH = -(-EPW // CH)
EPAD = NCH * CH
NP = 10112
RPS = NP // NS

_F32 = jnp.float32

_mesh = plsc.VectorSubcoreMesh(core_axis_name="c", subcore_axis_name="s")

_no_layout_cp = pltpu.CompilerParams()
if "needs_layout_passes" in pltpu.CompilerParams.__dataclass_fields__:
    _no_layout_cp = dataclasses.replace(_no_layout_cp,
                                        needs_layout_passes=False)


@functools.partial(
    pl.kernel,
    mesh=_mesh,
    out_type=jax.ShapeDtypeStruct((NW, NP), _F32),
    compiler_params=_no_layout_cp,
    scratch_types=[
        pltpu.VMEM((NCH, CH), jnp.int32),
        pltpu.VMEM((NP,), _F32),
    ],
)
def _deg_sc(dst_hbm, zeros_hbm, out_hbm, dstv, hist):
    c = lax.axis_index("c")
    s = lax.axis_index("s")
    wid = c * NS + s
    pltpu.sync_copy(zeros_hbm, hist)
    pltpu.sync_copy(dst_hbm.at[wid], dstv)
    ones16 = jnp.full((16,), 1.0, _F32)

    @pl.loop(0, NCH)
    def _(j):
        for k in range(CH // 16):
            d16 = dstv[j, pl.ds(k * 16, 16)]
            plsc.addupdate_scatter(hist, [d16], ones16)

    pltpu.sync_copy(hist, out_hbm.at[wid])


@functools.partial(
    pl.kernel,
    mesh=_mesh,
    out_type=jax.ShapeDtypeStruct((NC, NP, D), _F32),
    scratch_types=[
        pltpu.VMEM((2, CH), jnp.int32),
        pltpu.VMEM((2, CH), jnp.int32),
        pltpu.VMEM((CH, D), _F32),
        pltpu.VMEM((CH, D), _F32),
        pltpu.VMEM_SHARED((NP, D), _F32),
        pltpu.SemaphoreType.DMA,
        pltpu.SemaphoreType.DMA,
        pltpu.SemaphoreType.DMA,
        pltpu.SemaphoreType.DMA,
        pltpu.SemaphoreType.DMA,
        pltpu.SemaphoreType.DMA,
    ],
)
def _gs_sc(p_hbm, sidx_hbm, zeros_hbm, out_hbm,
           i0, i1, b0, b1, acc, gsem0, gsem1, isem0, isem1, ssem0, ssem1):
    c = lax.axis_index("c")
    s = lax.axis_index("s")
    wid = c * NS + s
    r0 = s * RPS
    pltpu.sync_copy(zeros_hbm.at[pl.ds(r0, RPS)], acc.at[pl.ds(r0, RPS)])
    plsc.subcore_barrier()

    def _idx(j, ibuf, sem):
        return pltpu.make_async_copy(sidx_hbm.at[wid, j], ibuf, sem)

    def _gather(ibuf, buf, sem):
        return pltpu.make_async_copy(p_hbm.at[ibuf.at[0]], buf, sem)

    class _scat:
        def __init__(self, buf, ibuf, sem):
            self.buf, self.ibuf, self.sem = buf, ibuf, sem

        def start(self):
            pltpu.async_copy(self.buf, acc.at[self.ibuf.at[1]], self.sem,
                             add=True)

        def wait(self):
            pltpu.make_async_copy(self.buf, acc.at[self.ibuf.at[1]],
                                  self.sem).wait()

    assert NCH % 2 == 1
    _idx(0, i0, isem0).start()
    _idx(0, i0, isem0).wait()
    _gather(i0, b0, gsem0).start()
    _idx(1, i1, isem1).start()

    @pl.loop(0, NCH - 1, step=2)
    def _(j):
        _idx(j + 1, i1, isem1).wait()
        _gather(i0, b0, gsem0).wait()
        _scat(b0, i0, ssem0).start()
        _gather(i1, b1, gsem1).start()
        _scat(b0, i0, ssem0).wait()
        _idx(j + 2, i0, isem0).start()
        _idx(j + 2, i0, isem0).wait()
        _gather(i1, b1, gsem1).wait()
        _scat(b1, i1, ssem1).start()
        _gather(i0, b0, gsem0).start()
        _scat(b1, i1, ssem1).wait()
        _idx(j + 3, i1, isem1).start()

    _gather(i0, b0, gsem0).wait()
    _scat(b0, i0, ssem0).start()
    _scat(b0, i0, ssem0).wait()
    _idx(NCH, i1, isem1).wait()

    plsc.subcore_barrier()
    pltpu.sync_copy(acc.at[pl.ds(r0, RPS)], out_hbm.at[c, pl.ds(r0, RPS)])


def _tc_mm(x, w):
    def body(x_ref, w_ref, o_ref):
        o_ref[...] = jnp.dot(x_ref[...], w_ref[...],
                             preferred_element_type=_F32,
                             precision=lax.Precision.HIGHEST)

    return pl.pallas_call(
        body, out_shape=jax.ShapeDtypeStruct((x.shape[0], w.shape[1]), _F32),
    )(x, w)


def _tc_scale(cnt, h1):
    def body(cnt_ref, h_ref, dinv_ref, p_ref):
        ones32 = jnp.ones((NW, 1), _F32)
        degc = lax.dot_general(cnt_ref[...], ones32, (((0,), (0,)), ((), ())),
                               preferred_element_type=_F32,
                               precision=lax.Precision.HIGHEST)
        deg = degc[0:N, :] + 1.0
        dinv = lax.rsqrt(deg)
        dinv_ref[...] = dinv
        p_ref[0:N, :] = h_ref[...] * dinv
        p_ref[N:NP, :] = jnp.zeros((NP - N, D), _F32)

    return pl.pallas_call(
        body,
        out_shape=[jax.ShapeDtypeStruct((N, 1), _F32),
                   jax.ShapeDtypeStruct((NP, D), _F32)],
    )(cnt, h1)


def _tc_mid(s1, p1, dinv, b, w):
    def body(s_ref, p_ref, dinv_ref, b_ref, w_ref, p2_ref):
        u = s_ref[0, 0:N, :] + s_ref[1, 0:N, :] + p_ref[0:N, :]
        h = jnp.maximum(u * dinv_ref[...] + b_ref[...], 0.0)
        h2 = jnp.dot(h, w_ref[...], preferred_element_type=_F32,
                     precision=lax.Precision.HIGHEST)
        p2_ref[0:N, :] = h2 * dinv_ref[...]
        p2_ref[N:NP, :] = jnp.zeros((NP - N, D), _F32)

    return pl.pallas_call(
        body, out_shape=jax.ShapeDtypeStruct((NP, D), _F32),
    )(s1, p1, dinv, b, w)


def _tc_final(s2, p2, dinv, b, batch2, fc_w, fc_b):
    def body(s_ref, p_ref, dinv_ref, b_ref, batch_ref, fcw_ref, fcb_ref,
             o_ref):
        u = s_ref[0, 0:N, :] + s_ref[1, 0:N, :] + p_ref[0:N, :]
        h = jnp.maximum(u * dinv_ref[...] + b_ref[...], 0.0)
        gids = lax.broadcasted_iota(jnp.int32, (N, G), 1)
        m = (batch_ref[...] == gids).astype(_F32)
        gsum = lax.dot_general(m, h, (((0,), (0,)), ((), ())),
                               preferred_element_type=_F32,
                               precision=lax.Precision.HIGHEST)
        counts = jnp.sum(m, axis=0)[:, None]
        mean = gsum / jnp.maximum(counts, 1.0)
        logits = jnp.dot(mean, fcw_ref[...], preferred_element_type=_F32,
                         precision=lax.Precision.HIGHEST) + fcb_ref[...]
        mx = jnp.max(logits, axis=1, keepdims=True)
        lse = jnp.log(jnp.sum(jnp.exp(logits - mx), axis=1,
                              keepdims=True)) + mx
        o_ref[...] = logits - lse

    return pl.pallas_call(
        body, out_shape=jax.ShapeDtypeStruct((G, DOUT), _F32),
    )(s2, p2, dinv, b, batch2, fc_w, fc_b)


def kernel(x, edge_index, batch, W1, b1, W2, b2, fc_W, fc_b):
    ei = edge_index.astype(jnp.int32)
    pad = jnp.full((NW, EPAD - EPW), N, jnp.int32)
    dst3 = jnp.concatenate([ei[1].reshape(NW, EPW), pad], axis=1)
    dst3 = dst3.reshape(NW, NCH, CH)
    src3 = jnp.concatenate([ei[0].reshape(NW, EPW), pad], axis=1)
    src3 = src3.reshape(NW, NCH, CH)
    sidx = jnp.stack([src3, dst3], axis=2)
    padc = jnp.full((NW, 2, 2, CH), N, jnp.int32)
    sidx = jnp.concatenate([sidx, padc], axis=1)

    zerosN = jnp.zeros((NP,), _F32)
    zerosD = jnp.zeros((NP, D), _F32)
    batch2 = batch.astype(jnp.int32).reshape(N, 1)
    b1r = b1.reshape(1, D)
    b2r = b2.reshape(1, D)
    fcbr = fc_b.reshape(1, DOUT)

    cnt = _deg_sc(dst3, zerosN)
    h1 = _tc_mm(x, W1)
    dinv, p1 = _tc_scale(cnt, h1)
    s1 = _gs_sc(p1, sidx, zerosD)
    p2 = _tc_mid(s1, p1, dinv, b1r, W2)
    s2 = _gs_sc(p2, sidx, zerosD)
    return _tc_final(s2, p2, dinv, b2r, batch2, fc_W, fcbr)
